# Initial kernel scaffold; baseline (speedup 1.0000x reference)
import functools

import jax
import jax.numpy as jnp
from jax import lax
from jax.experimental import pallas as pl
from jax.experimental.pallas import tpu as pltpu
from jax.experimental.pallas import tpu_sc as plsc

N_NODES = 10000
N_EDGES = 320000
D_IN = 128
D_OUT = 16

NC = 2
NS = 16
NW = NC * NS
CHUNK = 128
E_PER_W = -(-N_EDGES // (NW * CHUNK)) * CHUNK
NCHUNKS = E_PER_W // CHUNK
E_PAD = E_PER_W * NW
N_PAD = 10016
ROWS_PER_SUB = N_PAD // NS

ROW_BLOCK = 1000


def _proj_body(x_ref, wcat_ref, bcat_ref, zs_ref, y_ref):
    z = jnp.dot(x_ref[...], wcat_ref[...], preferred_element_type=jnp.float32)
    z = z + bcat_ref[...]
    zs_ref[...] = z[:, :D_OUT]
    y_ref[...] = z[:, D_OUT:]


def _combine_body(zs_ref, agg_ref, deg_ref, o_ref):
    agg = agg_ref[0] + agg_ref[1]
    deg = deg_ref[0] + deg_ref[1]
    o_ref[...] = jax.nn.sigmoid(zs_ref[...] + agg / jnp.maximum(deg, 1.0))


def _sc_segment_sum(y, srcp, dstp):
    mesh = plsc.VectorSubcoreMesh(core_axis_name="c", subcore_axis_name="s")

    @functools.partial(
        pl.kernel,
        out_type=(
            jax.ShapeDtypeStruct((NC, N_PAD, D_OUT), jnp.float32),
            jax.ShapeDtypeStruct((NC, N_PAD, D_OUT), jnp.float32),
        ),
        mesh=mesh,
        scratch_types=[
            pltpu.VMEM((NCHUNKS, CHUNK), jnp.int32),
            pltpu.VMEM((NCHUNKS, CHUNK), jnp.int32),
            pltpu.VMEM((CHUNK, D_OUT), jnp.float32),
            pltpu.VMEM((CHUNK, D_OUT), jnp.float32),
            pltpu.VMEM((ROWS_PER_SUB, D_OUT), jnp.float32),
            pltpu.VMEM_SHARED((N_PAD, D_OUT), jnp.float32),
            pltpu.VMEM_SHARED((N_PAD, D_OUT), jnp.float32),
        ],
    )
    def sc_kernel(y_hbm, src_hbm, dst_hbm, agg_out, deg_out,
                  src_v, dst_v, rows_v, ones_v, zbuf, agg_sh, deg_sh):
        cid = lax.axis_index("c")
        sid = lax.axis_index("s")
        wid = sid * NC + cid
        base = sid * ROWS_PER_SUB

        @pl.loop(0, CHUNK)
        def _(i):
            ones_v.at[i][...] = jnp.full((D_OUT,), 1.0, jnp.float32)

        @pl.loop(0, ROWS_PER_SUB)
        def _(i):
            zbuf.at[i][...] = jnp.zeros((D_OUT,), jnp.float32)

        pltpu.sync_copy(zbuf, agg_sh.at[pl.ds(base, ROWS_PER_SUB)])
        pltpu.sync_copy(zbuf, deg_sh.at[pl.ds(base, ROWS_PER_SUB)])
        pltpu.sync_copy(src_hbm.at[wid], src_v)
        pltpu.sync_copy(dst_hbm.at[wid], dst_v)

        plsc.subcore_barrier()

        @pl.loop(0, NCHUNKS)
        def _(j):
            pltpu.sync_copy(y_hbm.at[src_v.at[j]], rows_v)
            pltpu.sync_copy(rows_v, agg_sh.at[dst_v.at[j]], add=True)
            pltpu.sync_copy(ones_v, deg_sh.at[dst_v.at[j]], add=True)

        plsc.subcore_barrier()

        pltpu.sync_copy(agg_sh.at[pl.ds(base, ROWS_PER_SUB)],
                        agg_out.at[cid].at[pl.ds(base, ROWS_PER_SUB)])
        pltpu.sync_copy(deg_sh.at[pl.ds(base, ROWS_PER_SUB)],
                        deg_out.at[cid].at[pl.ds(base, ROWS_PER_SUB)])

    return sc_kernel(y, srcp, dstp)


def kernel(in_feat, edge_index, W_self, W_neigh, b):
    src = edge_index[0].astype(jnp.int32)
    dst = edge_index[1].astype(jnp.int32)
    pad = E_PAD - N_EDGES
    srcp = jnp.concatenate([src, jnp.zeros((pad,), jnp.int32)])
    dstp = jnp.concatenate([dst, jnp.full((pad,), N_NODES, jnp.int32)])
    srcp = srcp.reshape(NW, NCHUNKS, CHUNK)
    dstp = dstp.reshape(NW, NCHUNKS, CHUNK)
    wcat = jnp.concatenate([W_self, W_neigh], axis=1)
    bcat = jnp.concatenate([b, jnp.zeros((D_OUT,), jnp.float32)]).reshape(1, 2 * D_OUT)

    grid = (N_NODES // ROW_BLOCK,)
    zs, y = pl.pallas_call(
        _proj_body,
        grid=grid,
        in_specs=[
            pl.BlockSpec((ROW_BLOCK, D_IN), lambda i: (i, 0)),
            pl.BlockSpec((D_IN, 2 * D_OUT), lambda i: (0, 0)),
            pl.BlockSpec((1, 2 * D_OUT), lambda i: (0, 0)),
        ],
        out_specs=[
            pl.BlockSpec((ROW_BLOCK, D_OUT), lambda i: (i, 0)),
            pl.BlockSpec((ROW_BLOCK, D_OUT), lambda i: (i, 0)),
        ],
        out_shape=[
            jax.ShapeDtypeStruct((N_NODES, D_OUT), jnp.float32),
            jax.ShapeDtypeStruct((N_NODES, D_OUT), jnp.float32),
        ],
    )(in_feat, wcat, bcat)

    agg, deg = _sc_segment_sum(y, srcp, dstp)

    out = pl.pallas_call(
        _combine_body,
        grid=grid,
        in_specs=[
            pl.BlockSpec((ROW_BLOCK, D_OUT), lambda i: (i, 0)),
            pl.BlockSpec((NC, ROW_BLOCK, D_OUT), lambda i: (0, i, 0)),
            pl.BlockSpec((NC, ROW_BLOCK, D_OUT), lambda i: (0, i, 0)),
        ],
        out_specs=pl.BlockSpec((ROW_BLOCK, D_OUT), lambda i: (i, 0)),
        out_shape=jax.ShapeDtypeStruct((N_NODES, D_OUT), jnp.float32),
    )(zs, agg, deg)
    return out


# speedup vs baseline: 12.5264x; 12.5264x over previous
import functools

import jax
import jax.numpy as jnp
from jax import lax
from jax.experimental import pallas as pl
from jax.experimental.pallas import tpu as pltpu
from jax.experimental.pallas import tpu_sc as plsc

N_NODES = 10000
N_EDGES = 320000
D_IN = 128
D_OUT = 16

NC = 2
NS = 16
NW = NC * NS
CHUNK = 128
E_PER_W = -(-N_EDGES // (NW * CHUNK)) * CHUNK
NCHUNKS = E_PER_W // CHUNK
E_PAD = E_PER_W * NW
N_PAD = 10112
ROWS_PER_SUB = N_PAD // NS

ROW_BLOCK = 1000


def _proj_body(x_ref, wcat_ref, bcat_ref, zs_ref, y_ref):
    z = jnp.dot(x_ref[...], wcat_ref[...], preferred_element_type=jnp.float32)
    z = z + bcat_ref[...]
    zs_ref[...] = z[:, :D_OUT]
    y_ref[...] = z[:, D_OUT:]


def _combine_body(zs_ref, agg_ref, deg_ref, o_ref):
    agg = agg_ref[0] + agg_ref[1]
    deg = deg_ref[0] + deg_ref[1]
    o_ref[...] = jax.nn.sigmoid(zs_ref[...] + agg / jnp.maximum(deg, 1.0))


def _sc_segment_sum(y, srcp, dstp):
    mesh = plsc.VectorSubcoreMesh(core_axis_name="c", subcore_axis_name="s")

    @functools.partial(
        pl.kernel,
        out_type=(
            jax.ShapeDtypeStruct((NC, N_PAD, D_OUT), jnp.float32),
            jax.ShapeDtypeStruct((NC, N_PAD, D_OUT), jnp.float32),
        ),
        mesh=mesh,
        compiler_params=pltpu.CompilerParams(use_tc_tiling_on_sc=False),
        scratch_types=[
            pltpu.VMEM((NCHUNKS, CHUNK), jnp.int32),
            pltpu.VMEM((NCHUNKS, CHUNK), jnp.int32),
            pltpu.VMEM((CHUNK, D_OUT), jnp.float32),
            pltpu.VMEM((CHUNK, D_OUT), jnp.float32),
            pltpu.VMEM((ROWS_PER_SUB, D_OUT), jnp.float32),
            pltpu.VMEM_SHARED((N_PAD, D_OUT), jnp.float32),
            pltpu.VMEM_SHARED((N_PAD, D_OUT), jnp.float32),
        ],
    )
    def sc_kernel(y_hbm, src_hbm, dst_hbm, agg_out, deg_out,
                  src_v, dst_v, rows_v, ones_v, zbuf, agg_sh, deg_sh):
        cid = lax.axis_index("c")
        sid = lax.axis_index("s")
        wid = sid * NC + cid
        base = sid * ROWS_PER_SUB

        @pl.loop(0, CHUNK)
        def _(i):
            ones_v.at[i][...] = jnp.full((D_OUT,), 1.0, jnp.float32)

        @pl.loop(0, ROWS_PER_SUB)
        def _(i):
            zbuf.at[i][...] = jnp.zeros((D_OUT,), jnp.float32)

        pltpu.sync_copy(zbuf, agg_sh.at[pl.ds(base, ROWS_PER_SUB)])
        pltpu.sync_copy(zbuf, deg_sh.at[pl.ds(base, ROWS_PER_SUB)])
        pltpu.sync_copy(src_hbm.at[wid], src_v)
        pltpu.sync_copy(dst_hbm.at[wid], dst_v)

        plsc.subcore_barrier()

        @pl.loop(0, NCHUNKS)
        def _(j):
            pltpu.sync_copy(y_hbm.at[src_v.at[j]], rows_v)
            pltpu.sync_copy(rows_v, agg_sh.at[dst_v.at[j]], add=True)
            pltpu.sync_copy(ones_v, deg_sh.at[dst_v.at[j]], add=True)

        plsc.subcore_barrier()

        pltpu.sync_copy(agg_sh.at[pl.ds(base, ROWS_PER_SUB)],
                        agg_out.at[cid].at[pl.ds(base, ROWS_PER_SUB)])
        pltpu.sync_copy(deg_sh.at[pl.ds(base, ROWS_PER_SUB)],
                        deg_out.at[cid].at[pl.ds(base, ROWS_PER_SUB)])

    return sc_kernel(y, srcp, dstp)


def kernel(in_feat, edge_index, W_self, W_neigh, b):
    src = edge_index[0].astype(jnp.int32)
    dst = edge_index[1].astype(jnp.int32)
    pad = E_PAD - N_EDGES
    srcp = jnp.concatenate([src, jnp.zeros((pad,), jnp.int32)])
    dstp = jnp.concatenate([dst, jnp.full((pad,), N_NODES, jnp.int32)])
    srcp = srcp.reshape(NW, NCHUNKS, CHUNK)
    dstp = dstp.reshape(NW, NCHUNKS, CHUNK)
    wcat = jnp.concatenate([W_self, W_neigh], axis=1)
    bcat = jnp.concatenate([b, jnp.zeros((D_OUT,), jnp.float32)]).reshape(1, 2 * D_OUT)

    grid = (N_NODES // ROW_BLOCK,)
    zs, y = pl.pallas_call(
        _proj_body,
        grid=grid,
        in_specs=[
            pl.BlockSpec((ROW_BLOCK, D_IN), lambda i: (i, 0)),
            pl.BlockSpec((D_IN, 2 * D_OUT), lambda i: (0, 0)),
            pl.BlockSpec((1, 2 * D_OUT), lambda i: (0, 0)),
        ],
        out_specs=[
            pl.BlockSpec((ROW_BLOCK, D_OUT), lambda i: (i, 0)),
            pl.BlockSpec((ROW_BLOCK, D_OUT), lambda i: (i, 0)),
        ],
        out_shape=[
            jax.ShapeDtypeStruct((N_NODES, D_OUT), jnp.float32),
            jax.ShapeDtypeStruct((N_NODES, D_OUT), jnp.float32),
        ],
    )(in_feat, wcat, bcat)

    agg, deg = _sc_segment_sum(y, srcp, dstp)

    out = pl.pallas_call(
        _combine_body,
        grid=grid,
        in_specs=[
            pl.BlockSpec((ROW_BLOCK, D_OUT), lambda i: (i, 0)),
            pl.BlockSpec((NC, ROW_BLOCK, D_OUT), lambda i: (0, i, 0)),
            pl.BlockSpec((NC, ROW_BLOCK, D_OUT), lambda i: (0, i, 0)),
        ],
        out_specs=pl.BlockSpec((ROW_BLOCK, D_OUT), lambda i: (i, 0)),
        out_shape=jax.ShapeDtypeStruct((N_NODES, D_OUT), jnp.float32),
    )(zs, agg, deg)
    return out


# speedup vs baseline: 15.3423x; 1.2248x over previous
import functools

import jax
import jax.numpy as jnp
from jax import lax
from jax.experimental import pallas as pl
from jax.experimental.pallas import tpu as pltpu
from jax.experimental.pallas import tpu_sc as plsc

N_NODES = 10000
N_EDGES = 320000
D_IN = 128
D_OUT = 16

NC = 2
NS = 16
NW = NC * NS
CHUNK = 128
NBUF = 4
NCHUNKS = 80
E_PER_W = NCHUNKS * CHUNK
E_PAD = E_PER_W * NW
GROUPS = NCHUNKS // NBUF
N_PAD = 10112
ROWS_PER_SUB = N_PAD // NS

ROW_BLOCK = 1000


def _proj_body(x_ref, wcat_ref, bcat_ref, zs_ref, y_ref):
    z = jnp.dot(x_ref[...], wcat_ref[...], preferred_element_type=jnp.float32)
    z = z + bcat_ref[...]
    zs_ref[...] = z[:, :D_OUT]
    y_ref[...] = z[:, D_OUT:]


def _combine_body(zs_ref, agg_ref, deg_ref, o_ref):
    agg = agg_ref[0] + agg_ref[1]
    deg = deg_ref[0] + deg_ref[1]
    o_ref[...] = jax.nn.sigmoid(zs_ref[...] + agg / jnp.maximum(deg, 1.0))


def _sc_segment_sum(y, srcp, dstp):
    mesh = plsc.VectorSubcoreMesh(core_axis_name="c", subcore_axis_name="s")

    @functools.partial(
        pl.kernel,
        out_type=(
            jax.ShapeDtypeStruct((NC, N_PAD, D_OUT), jnp.float32),
            jax.ShapeDtypeStruct((NC, N_PAD, D_OUT), jnp.float32),
        ),
        mesh=mesh,
        compiler_params=pltpu.CompilerParams(use_tc_tiling_on_sc=False),
        scratch_types=[
            pltpu.VMEM((NCHUNKS, CHUNK), jnp.int32),
            pltpu.VMEM((NCHUNKS, CHUNK), jnp.int32),
            pltpu.VMEM((NBUF, CHUNK, D_OUT), jnp.float32),
            pltpu.VMEM((CHUNK, D_OUT), jnp.float32),
            pltpu.VMEM((ROWS_PER_SUB, D_OUT), jnp.float32),
            pltpu.VMEM_SHARED((N_PAD, D_OUT), jnp.float32),
            pltpu.VMEM_SHARED((N_PAD, D_OUT), jnp.float32),
            [pltpu.SemaphoreType.DMA] * NBUF,
            pltpu.SemaphoreType.DMA,
            pltpu.SemaphoreType.DMA,
        ],
    )
    def sc_kernel(y_hbm, src_hbm, dst_hbm, agg_out, deg_out,
                  src_v, dst_v, rows_v, ones_v, zbuf, agg_sh, deg_sh,
                  sem_g, sem_s, sem_d):
        cid = lax.axis_index("c")
        sid = lax.axis_index("s")
        wid = sid * NC + cid
        base = sid * ROWS_PER_SUB

        @pl.loop(0, CHUNK)
        def _(i):
            ones_v.at[i][...] = jnp.full((D_OUT,), 1.0, jnp.float32)

        @pl.loop(0, ROWS_PER_SUB)
        def _(i):
            zbuf.at[i][...] = jnp.zeros((D_OUT,), jnp.float32)

        pltpu.sync_copy(zbuf, agg_sh.at[pl.ds(base, ROWS_PER_SUB)])
        pltpu.sync_copy(zbuf, deg_sh.at[pl.ds(base, ROWS_PER_SUB)])
        pltpu.sync_copy(src_hbm.at[wid], src_v)
        pltpu.sync_copy(dst_hbm.at[wid], dst_v)

        plsc.subcore_barrier()

        for b in range(NBUF):
            pltpu.async_copy(y_hbm.at[src_v.at[b]], rows_v.at[b], sem_g[b])

        @pl.loop(0, GROUPS)
        def _(g):
            j0 = g * NBUF
            for b in range(NBUF):
                j = j0 + b
                pltpu.make_async_copy(
                    y_hbm.at[src_v.at[b]], rows_v.at[b], sem_g[b]).wait()
                hs = pltpu.async_copy(
                    rows_v.at[b], agg_sh.at[dst_v.at[j]], sem_s, add=True)
                hd = pltpu.async_copy(
                    ones_v, deg_sh.at[dst_v.at[j]], sem_d, add=True)
                hs.wait()
                hd.wait()

                @pl.when(j + NBUF < NCHUNKS)
                def _():
                    pltpu.async_copy(
                        y_hbm.at[src_v.at[j + NBUF]], rows_v.at[b], sem_g[b])

        plsc.subcore_barrier()

        pltpu.sync_copy(agg_sh.at[pl.ds(base, ROWS_PER_SUB)],
                        agg_out.at[cid].at[pl.ds(base, ROWS_PER_SUB)])
        pltpu.sync_copy(deg_sh.at[pl.ds(base, ROWS_PER_SUB)],
                        deg_out.at[cid].at[pl.ds(base, ROWS_PER_SUB)])

    return sc_kernel(y, srcp, dstp)


def kernel(in_feat, edge_index, W_self, W_neigh, b):
    src = edge_index[0].astype(jnp.int32)
    dst = edge_index[1].astype(jnp.int32)
    pad = E_PAD - N_EDGES
    srcp = jnp.concatenate([src, jnp.zeros((pad,), jnp.int32)])
    dstp = jnp.concatenate([dst, jnp.full((pad,), N_NODES, jnp.int32)])
    srcp = srcp.reshape(NW, NCHUNKS, CHUNK)
    dstp = dstp.reshape(NW, NCHUNKS, CHUNK)
    wcat = jnp.concatenate([W_self, W_neigh], axis=1)
    bcat = jnp.concatenate([b, jnp.zeros((D_OUT,), jnp.float32)]).reshape(1, 2 * D_OUT)

    grid = (N_NODES // ROW_BLOCK,)
    zs, y = pl.pallas_call(
        _proj_body,
        grid=grid,
        in_specs=[
            pl.BlockSpec((ROW_BLOCK, D_IN), lambda i: (i, 0)),
            pl.BlockSpec((D_IN, 2 * D_OUT), lambda i: (0, 0)),
            pl.BlockSpec((1, 2 * D_OUT), lambda i: (0, 0)),
        ],
        out_specs=[
            pl.BlockSpec((ROW_BLOCK, D_OUT), lambda i: (i, 0)),
            pl.BlockSpec((ROW_BLOCK, D_OUT), lambda i: (i, 0)),
        ],
        out_shape=[
            jax.ShapeDtypeStruct((N_NODES, D_OUT), jnp.float32),
            jax.ShapeDtypeStruct((N_NODES, D_OUT), jnp.float32),
        ],
    )(in_feat, wcat, bcat)

    agg, deg = _sc_segment_sum(y, srcp, dstp)

    out = pl.pallas_call(
        _combine_body,
        grid=grid,
        in_specs=[
            pl.BlockSpec((ROW_BLOCK, D_OUT), lambda i: (i, 0)),
            pl.BlockSpec((NC, ROW_BLOCK, D_OUT), lambda i: (0, i, 0)),
            pl.BlockSpec((NC, ROW_BLOCK, D_OUT), lambda i: (0, i, 0)),
        ],
        out_specs=pl.BlockSpec((ROW_BLOCK, D_OUT), lambda i: (i, 0)),
        out_shape=jax.ShapeDtypeStruct((N_NODES, D_OUT), jnp.float32),
    )(zs, agg, deg)
    return out


# speedup vs baseline: 16.1493x; 1.0526x over previous
import functools

import jax
import jax.numpy as jnp
from jax import lax
from jax.experimental import pallas as pl
from jax.experimental.pallas import tpu as pltpu
from jax.experimental.pallas import tpu_sc as plsc

N_NODES = 10000
N_EDGES = 320000
D_IN = 128
D_OUT = 16

NC = 2
NS = 16
NW = NC * NS
CHUNK = 128
NBUF = 8
LEAD = 4
NCHUNKS = 80
E_PER_W = NCHUNKS * CHUNK
E_PAD = E_PER_W * NW
GROUPS = NCHUNKS // NBUF
N_PAD = 10112
ROWS_PER_SUB = N_PAD // NS

ROW_BLOCK = 1000


def _proj_body(x_ref, wcat_ref, bcat_ref, zs_ref, y_ref):
    z = jnp.dot(x_ref[...], wcat_ref[...], preferred_element_type=jnp.float32)
    z = z + bcat_ref[...]
    zs_ref[...] = z[:, :D_OUT]
    y_ref[...] = z[:, D_OUT:]


def _combine_body(zs_ref, agg_ref, deg_ref, o_ref):
    agg = agg_ref[0] + agg_ref[1]
    deg = deg_ref[0] + deg_ref[1]
    o_ref[...] = jax.nn.sigmoid(zs_ref[...] + agg / jnp.maximum(deg, 1.0))


def _sc_segment_sum(y, eidx):
    mesh = plsc.VectorSubcoreMesh(core_axis_name="c", subcore_axis_name="s")

    @functools.partial(
        pl.kernel,
        out_type=(
            jax.ShapeDtypeStruct((NC, N_PAD, D_OUT), jnp.float32),
            jax.ShapeDtypeStruct((NC, N_PAD, D_OUT), jnp.float32),
        ),
        mesh=mesh,
        compiler_params=pltpu.CompilerParams(use_tc_tiling_on_sc=False),
        scratch_types=[
            pltpu.VMEM((NCHUNKS, CHUNK), jnp.int32),
            pltpu.VMEM((NCHUNKS, CHUNK), jnp.int32),
            pltpu.VMEM((NBUF, CHUNK, D_OUT), jnp.float32),
            pltpu.VMEM((CHUNK, D_OUT), jnp.float32),
            pltpu.VMEM((ROWS_PER_SUB, D_OUT), jnp.float32),
            pltpu.VMEM_SHARED((N_PAD, D_OUT), jnp.float32),
            pltpu.VMEM_SHARED((N_PAD, D_OUT), jnp.float32),
            [pltpu.SemaphoreType.DMA] * NBUF,
            [pltpu.SemaphoreType.DMA] * NBUF,
            [pltpu.SemaphoreType.DMA] * NBUF,
        ],
    )
    def sc_kernel(y_hbm, eidx_hbm, agg_out, deg_out,
                  src_v, dst_v, rows_v, ones_v, zbuf, agg_sh, deg_sh,
                  sem_g, sem_s, sem_d):
        cid = lax.axis_index("c")
        sid = lax.axis_index("s")
        wid = sid * NC + cid
        base = sid * ROWS_PER_SUB

        @pl.loop(0, CHUNK)
        def _(i):
            ones_v.at[i][...] = jnp.full((D_OUT,), 1.0, jnp.float32)

        @pl.loop(0, ROWS_PER_SUB)
        def _(i):
            zbuf.at[i][...] = jnp.zeros((D_OUT,), jnp.float32)

        pltpu.sync_copy(zbuf, agg_sh.at[pl.ds(base, ROWS_PER_SUB)])
        pltpu.sync_copy(zbuf, deg_sh.at[pl.ds(base, ROWS_PER_SUB)])
        pltpu.sync_copy(eidx_hbm.at[0].at[wid], src_v)
        pltpu.sync_copy(eidx_hbm.at[1].at[wid], dst_v)

        plsc.subcore_barrier()

        def slot(j, b, steady):
            pb = (b + LEAD) % NBUF
            if steady:
                pltpu.make_async_copy(
                    rows_v.at[pb], agg_sh.at[dst_v.at[0]], sem_s[pb]).wait()
                pltpu.make_async_copy(
                    ones_v, deg_sh.at[dst_v.at[0]], sem_d[pb]).wait()

            if isinstance(j, int):
                pltpu.async_copy(
                    y_hbm.at[src_v.at[j + LEAD]], rows_v.at[pb], sem_g[pb])
            else:
                @pl.when(j + LEAD < NCHUNKS)
                def _():
                    pltpu.async_copy(
                        y_hbm.at[src_v.at[j + LEAD]], rows_v.at[pb], sem_g[pb])

            pltpu.make_async_copy(
                y_hbm.at[src_v.at[0]], rows_v.at[b], sem_g[b]).wait()
            pltpu.async_copy(
                rows_v.at[b], agg_sh.at[dst_v.at[j]], sem_s[b], add=True)
            pltpu.async_copy(
                ones_v, deg_sh.at[dst_v.at[j]], sem_d[b], add=True)

        for b in range(LEAD):
            pltpu.async_copy(y_hbm.at[src_v.at[b]], rows_v.at[b], sem_g[b])
        for j in range(NBUF):
            slot(j, j, steady=j >= LEAD)

        @pl.loop(1, GROUPS)
        def _(g):
            j0 = g * NBUF
            for b in range(NBUF):
                slot(j0 + b, b, steady=True)

        for b in range(NBUF - LEAD, NBUF):
            pltpu.make_async_copy(
                rows_v.at[b], agg_sh.at[dst_v.at[0]], sem_s[b]).wait()
            pltpu.make_async_copy(
                ones_v, deg_sh.at[dst_v.at[0]], sem_d[b]).wait()

        plsc.subcore_barrier()

        pltpu.sync_copy(agg_sh.at[pl.ds(base, ROWS_PER_SUB)],
                        agg_out.at[cid].at[pl.ds(base, ROWS_PER_SUB)])
        pltpu.sync_copy(deg_sh.at[pl.ds(base, ROWS_PER_SUB)],
                        deg_out.at[cid].at[pl.ds(base, ROWS_PER_SUB)])

    return sc_kernel(y, eidx)


def kernel(in_feat, edge_index, W_self, W_neigh, b):
    eidx = edge_index.astype(jnp.int32)
    pad = E_PAD - N_EDGES
    filler = jnp.broadcast_to(
        jnp.array([[0], [N_NODES]], jnp.int32), (2, pad))
    eidx_p = jnp.concatenate([eidx, filler], axis=1)
    eidx_p = eidx_p.reshape(2, NW, NCHUNKS, CHUNK)
    wcat = jnp.concatenate([W_self, W_neigh], axis=1)
    bcat = jnp.concatenate([b, jnp.zeros((D_OUT,), jnp.float32)]).reshape(1, 2 * D_OUT)

    grid = (N_NODES // ROW_BLOCK,)
    zs, y = pl.pallas_call(
        _proj_body,
        grid=grid,
        in_specs=[
            pl.BlockSpec((ROW_BLOCK, D_IN), lambda i: (i, 0)),
            pl.BlockSpec((D_IN, 2 * D_OUT), lambda i: (0, 0)),
            pl.BlockSpec((1, 2 * D_OUT), lambda i: (0, 0)),
        ],
        out_specs=[
            pl.BlockSpec((ROW_BLOCK, D_OUT), lambda i: (i, 0)),
            pl.BlockSpec((ROW_BLOCK, D_OUT), lambda i: (i, 0)),
        ],
        out_shape=[
            jax.ShapeDtypeStruct((N_NODES, D_OUT), jnp.float32),
            jax.ShapeDtypeStruct((N_NODES, D_OUT), jnp.float32),
        ],
    )(in_feat, wcat, bcat)

    agg, deg = _sc_segment_sum(y, eidx_p)

    out = pl.pallas_call(
        _combine_body,
        grid=grid,
        in_specs=[
            pl.BlockSpec((ROW_BLOCK, D_OUT), lambda i: (i, 0)),
            pl.BlockSpec((NC, ROW_BLOCK, D_OUT), lambda i: (0, i, 0)),
            pl.BlockSpec((NC, ROW_BLOCK, D_OUT), lambda i: (0, i, 0)),
        ],
        out_specs=pl.BlockSpec((ROW_BLOCK, D_OUT), lambda i: (i, 0)),
        out_shape=jax.ShapeDtypeStruct((N_NODES, D_OUT), jnp.float32),
    )(zs, agg, deg)
    return out


# speedup vs baseline: 19.3231x; 1.1965x over previous
import functools

import jax
import jax.numpy as jnp
from jax import lax
from jax.experimental import pallas as pl
from jax.experimental.pallas import tpu as pltpu
from jax.experimental.pallas import tpu_sc as plsc

N_NODES = 10000
N_EDGES = 320000
D_IN = 128
D_OUT = 16

NC = 2
NS = 16
NW = NC * NS
CHUNK = 128
NBUF = 8
LEAD = 4
NCHUNKS = 80
E_PER_W = NCHUNKS * CHUNK
E_PAD = E_PER_W * NW
ROWS_IDX = E_PAD // CHUNK
N_PAD = 10112
ROWS_PER_SUB = N_PAD // NS

GRID = 10
ROW_BLOCK = 1024
FLAT_BLOCK = ROW_BLOCK * D_OUT // 128
EDGE_BLOCK = E_PAD // GRID
IDX_BLOCK = EDGE_BLOCK // CHUNK
N_FLAT = N_NODES * D_OUT // 128
P_FLAT = N_PAD * D_OUT // 128


def _prep_body(xf_ref, wbig_ref, bbig_ref, eidx_ref, zs_ref, y_ref, idx_ref):
    i = pl.program_id(0)
    z = jnp.dot(xf_ref[...], wbig_ref[...], preferred_element_type=jnp.float32)
    z = z + bbig_ref[...]
    zs_ref[...] = z[:, :128]
    y_ref[...] = z[:, 128:]
    pos = (i * EDGE_BLOCK
           + lax.broadcasted_iota(jnp.int32, (IDX_BLOCK, CHUNK), 0) * CHUNK
           + lax.broadcasted_iota(jnp.int32, (IDX_BLOCK, CHUNK), 1))
    valid = pos < N_EDGES
    src = eidx_ref[0:1, :].reshape(IDX_BLOCK, CHUNK)
    dst = eidx_ref[1:2, :].reshape(IDX_BLOCK, CHUNK)
    idx_ref[0] = jnp.where(valid, src, 0)
    idx_ref[1] = jnp.where(valid, dst, N_NODES)


def _combine_body(zs_ref, agg_ref, deg_ref, o_ref):
    agg = agg_ref[0] + agg_ref[1]
    deg = deg_ref[0] + deg_ref[1]
    o_ref[...] = jax.nn.sigmoid(zs_ref[...] + agg / jnp.maximum(deg, 1.0))


def _sc_segment_sum(y, idx):
    mesh = plsc.VectorSubcoreMesh(core_axis_name="c", subcore_axis_name="s")

    @functools.partial(
        pl.kernel,
        out_type=(
            jax.ShapeDtypeStruct((NC, N_PAD, D_OUT), jnp.float32),
            jax.ShapeDtypeStruct((NC, N_PAD, D_OUT), jnp.float32),
        ),
        mesh=mesh,
        compiler_params=pltpu.CompilerParams(use_tc_tiling_on_sc=False),
        scratch_types=[
            pltpu.VMEM((NCHUNKS, CHUNK), jnp.int32),
            pltpu.VMEM((NCHUNKS, CHUNK), jnp.int32),
            pltpu.VMEM((NBUF, CHUNK, D_OUT), jnp.float32),
            pltpu.VMEM((CHUNK, D_OUT), jnp.float32),
            pltpu.VMEM((ROWS_PER_SUB, D_OUT), jnp.float32),
            pltpu.VMEM_SHARED((N_PAD, D_OUT), jnp.float32),
            pltpu.VMEM_SHARED((N_PAD, D_OUT), jnp.float32),
            [pltpu.SemaphoreType.DMA] * NBUF,
            [pltpu.SemaphoreType.DMA] * NBUF,
            [pltpu.SemaphoreType.DMA] * NBUF,
        ],
    )
    def sc_kernel(y_hbm, idx_hbm, agg_out, deg_out,
                  src_v, dst_v, rows_v, ones_v, zbuf, agg_sh, deg_sh,
                  sem_g, sem_s, sem_d):
        cid = lax.axis_index("c")
        sid = lax.axis_index("s")
        wid = sid * NC + cid
        base = sid * ROWS_PER_SUB

        @pl.loop(0, CHUNK)
        def _(i):
            ones_v.at[i][...] = jnp.full((D_OUT,), 1.0, jnp.float32)

        @pl.loop(0, ROWS_PER_SUB)
        def _(i):
            zbuf.at[i][...] = jnp.zeros((D_OUT,), jnp.float32)

        pltpu.sync_copy(zbuf, agg_sh.at[pl.ds(base, ROWS_PER_SUB)])
        pltpu.sync_copy(zbuf, deg_sh.at[pl.ds(base, ROWS_PER_SUB)])
        pltpu.sync_copy(idx_hbm.at[0].at[pl.ds(wid * NCHUNKS, NCHUNKS)], src_v)
        pltpu.sync_copy(idx_hbm.at[1].at[pl.ds(wid * NCHUNKS, NCHUNKS)], dst_v)

        plsc.subcore_barrier()

        def slot(j, b, steady):
            pb = (b + LEAD) % NBUF
            if steady:
                pltpu.make_async_copy(
                    rows_v.at[pb], agg_sh.at[dst_v.at[0]], sem_s[pb]).wait()
                pltpu.make_async_copy(
                    ones_v, deg_sh.at[dst_v.at[0]], sem_d[pb]).wait()

            if isinstance(j, int):
                pltpu.async_copy(
                    y_hbm.at[src_v.at[j + LEAD]], rows_v.at[pb], sem_g[pb])
            else:
                @pl.when(j + LEAD < NCHUNKS)
                def _():
                    pltpu.async_copy(
                        y_hbm.at[src_v.at[j + LEAD]], rows_v.at[pb], sem_g[pb])

            pltpu.make_async_copy(
                y_hbm.at[src_v.at[0]], rows_v.at[b], sem_g[b]).wait()
            pltpu.async_copy(
                rows_v.at[b], agg_sh.at[dst_v.at[j]], sem_s[b], add=True)
            pltpu.async_copy(
                ones_v, deg_sh.at[dst_v.at[j]], sem_d[b], add=True)

        for b in range(LEAD):
            pltpu.async_copy(y_hbm.at[src_v.at[b]], rows_v.at[b], sem_g[b])
        for j in range(NBUF):
            slot(j, j, steady=j >= LEAD)

        @pl.loop(1, NCHUNKS // NBUF)
        def _(g):
            j0 = g * NBUF
            for b in range(NBUF):
                slot(j0 + b, b, steady=True)

        for b in range(NBUF - LEAD, NBUF):
            pltpu.make_async_copy(
                rows_v.at[b], agg_sh.at[dst_v.at[0]], sem_s[b]).wait()
            pltpu.make_async_copy(
                ones_v, deg_sh.at[dst_v.at[0]], sem_d[b]).wait()

        plsc.subcore_barrier()

        pltpu.sync_copy(agg_sh.at[pl.ds(base, ROWS_PER_SUB)],
                        agg_out.at[cid].at[pl.ds(base, ROWS_PER_SUB)])
        pltpu.sync_copy(deg_sh.at[pl.ds(base, ROWS_PER_SUB)],
                        deg_out.at[cid].at[pl.ds(base, ROWS_PER_SUB)])

    return sc_kernel(y, idx)


def kernel(in_feat, edge_index, W_self, W_neigh, b):
    eidx = edge_index.astype(jnp.int32)
    xf = in_feat.reshape(N_FLAT, 8 * D_IN)
    eye8 = jnp.eye(8, dtype=jnp.float32)
    wbig = jnp.concatenate(
        [jnp.kron(eye8, W_self), jnp.kron(eye8, W_neigh)], axis=1)
    bbig = jnp.concatenate(
        [jnp.tile(b, 8), jnp.zeros((128,), jnp.float32)]).reshape(1, 256)

    zs_flat, y_flat, idx = pl.pallas_call(
        _prep_body,
        grid=(GRID,),
        in_specs=[
            pl.BlockSpec((FLAT_BLOCK, 8 * D_IN), lambda i: (i, 0)),
            pl.BlockSpec((8 * D_IN, 256), lambda i: (0, 0)),
            pl.BlockSpec((1, 256), lambda i: (0, 0)),
            pl.BlockSpec((2, EDGE_BLOCK), lambda i: (0, i)),
        ],
        out_specs=[
            pl.BlockSpec((FLAT_BLOCK, 128), lambda i: (i, 0)),
            pl.BlockSpec((FLAT_BLOCK, 128), lambda i: (i, 0)),
            pl.BlockSpec((2, IDX_BLOCK, CHUNK), lambda i: (0, i, 0)),
        ],
        out_shape=[
            jax.ShapeDtypeStruct((N_FLAT, 128), jnp.float32),
            jax.ShapeDtypeStruct((N_FLAT, 128), jnp.float32),
            jax.ShapeDtypeStruct((2, ROWS_IDX, CHUNK), jnp.int32),
        ],
    )(xf, wbig, bbig, eidx)

    y = y_flat.reshape(N_NODES, D_OUT)
    agg, deg = _sc_segment_sum(y, idx)
    agg_flat = agg.reshape(NC, P_FLAT, 128)
    deg_flat = deg.reshape(NC, P_FLAT, 128)

    out_flat = pl.pallas_call(
        _combine_body,
        grid=(GRID,),
        in_specs=[
            pl.BlockSpec((FLAT_BLOCK, 128), lambda i: (i, 0)),
            pl.BlockSpec((NC, FLAT_BLOCK, 128), lambda i: (0, i, 0)),
            pl.BlockSpec((NC, FLAT_BLOCK, 128), lambda i: (0, i, 0)),
        ],
        out_specs=pl.BlockSpec((FLAT_BLOCK, 128), lambda i: (i, 0)),
        out_shape=jax.ShapeDtypeStruct((N_FLAT, 128), jnp.float32),
    )(zs_flat, agg_flat, deg_flat)
    return out_flat.reshape(N_NODES, D_OUT)


# speedup vs baseline: 19.3799x; 1.0029x over previous
import functools

import jax
import jax.numpy as jnp
from jax import lax
from jax.experimental import pallas as pl
from jax.experimental.pallas import tpu as pltpu
from jax.experimental.pallas import tpu_sc as plsc

N_NODES = 10000
N_EDGES = 320000
D_IN = 128
D_OUT = 16

NC = 2
NS = 16
NW = NC * NS
CHUNK = 128
NBUF = 10
LEAD = 5
NCHUNKS = 80
E_PER_W = NCHUNKS * CHUNK
E_PAD = E_PER_W * NW
ROWS_IDX = E_PAD // CHUNK
N_PAD = 10112
ROWS_PER_SUB = N_PAD // NS

GRID = 10
ROW_BLOCK = 1024
FLAT_BLOCK = ROW_BLOCK * D_OUT // 128
EDGE_BLOCK = E_PAD // GRID
IDX_BLOCK = EDGE_BLOCK // CHUNK
N_FLAT = N_NODES * D_OUT // 128
P_FLAT = N_PAD * D_OUT // 128


def _prep_body(xf_ref, wbig_ref, bbig_ref, eidx_ref, zs_ref, y_ref, idx_ref):
    i = pl.program_id(0)
    z = jnp.dot(xf_ref[...], wbig_ref[...], preferred_element_type=jnp.float32)
    z = z + bbig_ref[...]
    zs_ref[...] = z[:, :128]
    y_ref[...] = z[:, 128:]
    pos = (i * EDGE_BLOCK
           + lax.broadcasted_iota(jnp.int32, (IDX_BLOCK, CHUNK), 0) * CHUNK
           + lax.broadcasted_iota(jnp.int32, (IDX_BLOCK, CHUNK), 1))
    valid = pos < N_EDGES
    src = eidx_ref[0:1, :].reshape(IDX_BLOCK, CHUNK)
    dst = eidx_ref[1:2, :].reshape(IDX_BLOCK, CHUNK)
    idx_ref[0] = jnp.where(valid, src, 0)
    idx_ref[1] = jnp.where(valid, dst, N_NODES)


def _combine_body(zs_ref, agg_ref, deg_ref, o_ref):
    agg = agg_ref[0] + agg_ref[1]
    deg = deg_ref[0] + deg_ref[1]
    o_ref[...] = jax.nn.sigmoid(zs_ref[...] + agg / jnp.maximum(deg, 1.0))


def _sc_segment_sum(y, idx):
    mesh = plsc.VectorSubcoreMesh(core_axis_name="c", subcore_axis_name="s")

    @functools.partial(
        pl.kernel,
        out_type=(
            jax.ShapeDtypeStruct((NC, N_PAD, D_OUT), jnp.float32),
            jax.ShapeDtypeStruct((NC, N_PAD, D_OUT), jnp.float32),
        ),
        mesh=mesh,
        compiler_params=pltpu.CompilerParams(use_tc_tiling_on_sc=False),
        scratch_types=[
            pltpu.VMEM((NCHUNKS, CHUNK), jnp.int32),
            pltpu.VMEM((NCHUNKS, CHUNK), jnp.int32),
            pltpu.VMEM((NBUF, CHUNK, D_OUT), jnp.float32),
            pltpu.VMEM((CHUNK, D_OUT), jnp.float32),
            pltpu.VMEM((ROWS_PER_SUB, D_OUT), jnp.float32),
            pltpu.VMEM_SHARED((N_PAD, D_OUT), jnp.float32),
            pltpu.VMEM_SHARED((N_PAD, D_OUT), jnp.float32),
            [pltpu.SemaphoreType.DMA] * NBUF,
            [pltpu.SemaphoreType.DMA] * NBUF,
            [pltpu.SemaphoreType.DMA] * NBUF,
        ],
    )
    def sc_kernel(y_hbm, idx_hbm, agg_out, deg_out,
                  src_v, dst_v, rows_v, ones_v, zbuf, agg_sh, deg_sh,
                  sem_g, sem_s, sem_d):
        cid = lax.axis_index("c")
        sid = lax.axis_index("s")
        wid = sid * NC + cid
        base = sid * ROWS_PER_SUB

        @pl.loop(0, CHUNK)
        def _(i):
            ones_v.at[i][...] = jnp.full((D_OUT,), 1.0, jnp.float32)

        @pl.loop(0, ROWS_PER_SUB)
        def _(i):
            zbuf.at[i][...] = jnp.zeros((D_OUT,), jnp.float32)

        pltpu.sync_copy(zbuf, agg_sh.at[pl.ds(base, ROWS_PER_SUB)])
        pltpu.sync_copy(zbuf, deg_sh.at[pl.ds(base, ROWS_PER_SUB)])
        pltpu.sync_copy(idx_hbm.at[0].at[pl.ds(wid * NCHUNKS, NCHUNKS)], src_v)
        pltpu.sync_copy(idx_hbm.at[1].at[pl.ds(wid * NCHUNKS, NCHUNKS)], dst_v)

        plsc.subcore_barrier()

        def slot(j, b, steady):
            pb = (b + LEAD) % NBUF
            if steady:
                pltpu.make_async_copy(
                    rows_v.at[pb], agg_sh.at[dst_v.at[0]], sem_s[pb]).wait()
                pltpu.make_async_copy(
                    ones_v, deg_sh.at[dst_v.at[0]], sem_d[pb]).wait()

            if isinstance(j, int):
                pltpu.async_copy(
                    y_hbm.at[src_v.at[j + LEAD]], rows_v.at[pb], sem_g[pb])
            else:
                @pl.when(j + LEAD < NCHUNKS)
                def _():
                    pltpu.async_copy(
                        y_hbm.at[src_v.at[j + LEAD]], rows_v.at[pb], sem_g[pb])

            pltpu.make_async_copy(
                y_hbm.at[src_v.at[0]], rows_v.at[b], sem_g[b]).wait()
            pltpu.async_copy(
                rows_v.at[b], agg_sh.at[dst_v.at[j]], sem_s[b], add=True)
            pltpu.async_copy(
                ones_v, deg_sh.at[dst_v.at[j]], sem_d[b], add=True)

        for b in range(LEAD):
            pltpu.async_copy(y_hbm.at[src_v.at[b]], rows_v.at[b], sem_g[b])
        for j in range(NBUF):
            slot(j, j, steady=j >= LEAD)

        @pl.loop(1, NCHUNKS // NBUF)
        def _(g):
            j0 = g * NBUF
            for b in range(NBUF):
                slot(j0 + b, b, steady=True)

        for b in range(NBUF - LEAD, NBUF):
            pltpu.make_async_copy(
                rows_v.at[b], agg_sh.at[dst_v.at[0]], sem_s[b]).wait()
            pltpu.make_async_copy(
                ones_v, deg_sh.at[dst_v.at[0]], sem_d[b]).wait()

        plsc.subcore_barrier()

        pltpu.sync_copy(agg_sh.at[pl.ds(base, ROWS_PER_SUB)],
                        agg_out.at[cid].at[pl.ds(base, ROWS_PER_SUB)])
        pltpu.sync_copy(deg_sh.at[pl.ds(base, ROWS_PER_SUB)],
                        deg_out.at[cid].at[pl.ds(base, ROWS_PER_SUB)])

    return sc_kernel(y, idx)


def kernel(in_feat, edge_index, W_self, W_neigh, b):
    eidx = edge_index.astype(jnp.int32)
    xf = in_feat.reshape(N_FLAT, 8 * D_IN)
    eye8 = jnp.eye(8, dtype=jnp.float32)
    wbig = jnp.concatenate(
        [jnp.kron(eye8, W_self), jnp.kron(eye8, W_neigh)], axis=1)
    bbig = jnp.concatenate(
        [jnp.tile(b, 8), jnp.zeros((128,), jnp.float32)]).reshape(1, 256)

    zs_flat, y_flat, idx = pl.pallas_call(
        _prep_body,
        grid=(GRID,),
        in_specs=[
            pl.BlockSpec((FLAT_BLOCK, 8 * D_IN), lambda i: (i, 0)),
            pl.BlockSpec((8 * D_IN, 256), lambda i: (0, 0)),
            pl.BlockSpec((1, 256), lambda i: (0, 0)),
            pl.BlockSpec((2, EDGE_BLOCK), lambda i: (0, i)),
        ],
        out_specs=[
            pl.BlockSpec((FLAT_BLOCK, 128), lambda i: (i, 0)),
            pl.BlockSpec((FLAT_BLOCK, 128), lambda i: (i, 0)),
            pl.BlockSpec((2, IDX_BLOCK, CHUNK), lambda i: (0, i, 0)),
        ],
        out_shape=[
            jax.ShapeDtypeStruct((N_FLAT, 128), jnp.float32),
            jax.ShapeDtypeStruct((N_FLAT, 128), jnp.float32),
            jax.ShapeDtypeStruct((2, ROWS_IDX, CHUNK), jnp.int32),
        ],
    )(xf, wbig, bbig, eidx)

    y = y_flat.reshape(N_NODES, D_OUT)
    agg, deg = _sc_segment_sum(y, idx)
    agg_flat = agg.reshape(NC, P_FLAT, 128)
    deg_flat = deg.reshape(NC, P_FLAT, 128)

    out_flat = pl.pallas_call(
        _combine_body,
        grid=(GRID,),
        in_specs=[
            pl.BlockSpec((FLAT_BLOCK, 128), lambda i: (i, 0)),
            pl.BlockSpec((NC, FLAT_BLOCK, 128), lambda i: (0, i, 0)),
            pl.BlockSpec((NC, FLAT_BLOCK, 128), lambda i: (0, i, 0)),
        ],
        out_specs=pl.BlockSpec((FLAT_BLOCK, 128), lambda i: (i, 0)),
        out_shape=jax.ShapeDtypeStruct((N_FLAT, 128), jnp.float32),
    )(zs_flat, agg_flat, deg_flat)
    return out_flat.reshape(N_NODES, D_OUT)


# speedup vs baseline: 23.7707x; 1.2266x over previous
import functools

import jax
import jax.numpy as jnp
from jax import lax
from jax.experimental import pallas as pl
from jax.experimental.pallas import tpu as pltpu
from jax.experimental.pallas import tpu_sc as plsc

N_NODES = 10000
N_EDGES = 320000
D_IN = 128
D_OUT = 16

NC = 2
NS = 16
NW = NC * NS
CHUNK = 128
NBUF = 10
LEAD = 5
DIAG_NO_DEG = False
DIAG_NO_AGG = False
DIAG_NO_GATHER = False
NCHUNKS = 80
E_PER_W = NCHUNKS * CHUNK
E_PAD = E_PER_W * NW
ROWS_IDX = E_PAD // CHUNK
N_PAD = 10112
ROWS_PER_SUB = N_PAD // NS

GRID = 10
ROW_BLOCK = 1024
FLAT_BLOCK = ROW_BLOCK * D_OUT // 128
EDGE_BLOCK = E_PAD // GRID
IDX_BLOCK = EDGE_BLOCK // CHUNK
N_FLAT = N_NODES * D_OUT // 128
P_FLAT = N_PAD * D_OUT // 128


def _prep_body(xf_ref, wbig_ref, bbig_ref, eidx_ref, zs_ref, y_ref, idx_ref):
    i = pl.program_id(0)
    z = jnp.dot(xf_ref[...], wbig_ref[...], preferred_element_type=jnp.float32)
    z = z + bbig_ref[...]
    zs_ref[...] = z[:, :128]
    y_ref[...] = z[:, 128:]
    pos = (i * EDGE_BLOCK
           + lax.broadcasted_iota(jnp.int32, (IDX_BLOCK, CHUNK), 0) * CHUNK
           + lax.broadcasted_iota(jnp.int32, (IDX_BLOCK, CHUNK), 1))
    valid = pos < N_EDGES
    src = eidx_ref[0:1, :].reshape(IDX_BLOCK, CHUNK)
    dst = eidx_ref[1:2, :].reshape(IDX_BLOCK, CHUNK)
    idx_ref[0] = jnp.where(valid, src, 0)
    idx_ref[1] = jnp.where(valid, dst, N_NODES)


def _combine_body(zs_ref, agg_ref, deg_ref, o_ref):
    agg = agg_ref[0] + agg_ref[1]
    deg = deg_ref[0] + deg_ref[1]
    o_ref[...] = jax.nn.sigmoid(zs_ref[...] + agg / jnp.maximum(deg, 1.0))


def _sc_segment_sum(y, idx):
    mesh = plsc.VectorSubcoreMesh(core_axis_name="c", subcore_axis_name="s")

    @functools.partial(
        pl.kernel,
        out_type=(
            jax.ShapeDtypeStruct((NC, N_PAD, D_OUT), jnp.float32),
            jax.ShapeDtypeStruct((NC, N_PAD, D_OUT), jnp.float32),
        ),
        mesh=mesh,
        compiler_params=pltpu.CompilerParams(use_tc_tiling_on_sc=False),
        scratch_types=[
            pltpu.VMEM((NCHUNKS, CHUNK), jnp.int32),
            pltpu.VMEM((NCHUNKS, CHUNK), jnp.int32),
            pltpu.VMEM((NBUF, CHUNK, D_OUT), jnp.float32),
            pltpu.VMEM((CHUNK, D_OUT), jnp.float32),
            pltpu.VMEM((ROWS_PER_SUB, D_OUT), jnp.float32),
            pltpu.VMEM_SHARED((N_PAD, D_OUT), jnp.float32),
            pltpu.VMEM_SHARED((N_PAD, D_OUT), jnp.float32),
            pltpu.VMEM_SHARED((N_PAD, D_OUT), jnp.float32),
            [pltpu.SemaphoreType.DMA] * NBUF,
            [pltpu.SemaphoreType.DMA] * NBUF,
            [pltpu.SemaphoreType.DMA] * NBUF,
        ],
    )
    def sc_kernel(y_hbm, idx_hbm, agg_out, deg_out,
                  src_v, dst_v, rows_v, ones_v, zbuf, y_sh, agg_sh, deg_sh,
                  sem_g, sem_s, sem_d):
        cid = lax.axis_index("c")
        sid = lax.axis_index("s")
        wid = sid * NC + cid
        base = sid * ROWS_PER_SUB

        @pl.loop(0, CHUNK)
        def _(i):
            ones_v.at[i][...] = jnp.full((D_OUT,), 1.0, jnp.float32)

        @pl.loop(0, ROWS_PER_SUB)
        def _(i):
            zbuf.at[i][...] = jnp.zeros((D_OUT,), jnp.float32)

        pltpu.sync_copy(zbuf, agg_sh.at[pl.ds(base, ROWS_PER_SUB)])
        pltpu.sync_copy(zbuf, deg_sh.at[pl.ds(base, ROWS_PER_SUB)])
        pltpu.sync_copy(y_hbm.at[pl.ds(base, ROWS_PER_SUB)],
                        y_sh.at[pl.ds(base, ROWS_PER_SUB)])
        pltpu.sync_copy(idx_hbm.at[0].at[pl.ds(wid * NCHUNKS, NCHUNKS)], src_v)
        pltpu.sync_copy(idx_hbm.at[1].at[pl.ds(wid * NCHUNKS, NCHUNKS)], dst_v)

        plsc.subcore_barrier()

        def slot(j, b, steady):
            pb = (b + LEAD) % NBUF
            if steady:
                if not DIAG_NO_AGG:
                    pltpu.make_async_copy(
                        rows_v.at[pb], agg_sh.at[dst_v.at[0]], sem_s[pb]).wait()
                if not DIAG_NO_DEG:
                    pltpu.make_async_copy(
                        ones_v, deg_sh.at[dst_v.at[0]], sem_d[pb]).wait()

            if not DIAG_NO_GATHER:
                if isinstance(j, int):
                    pltpu.async_copy(
                        y_sh.at[src_v.at[j + LEAD]], rows_v.at[pb], sem_g[pb])
                else:
                    @pl.when(j + LEAD < NCHUNKS)
                    def _():
                        pltpu.async_copy(
                            y_sh.at[src_v.at[j + LEAD]], rows_v.at[pb], sem_g[pb])

                pltpu.make_async_copy(
                    y_sh.at[src_v.at[0]], rows_v.at[b], sem_g[b]).wait()
            if not DIAG_NO_AGG:
                pltpu.async_copy(
                    rows_v.at[b], agg_sh.at[dst_v.at[j]], sem_s[b], add=True)
            if not DIAG_NO_DEG:
                pltpu.async_copy(
                    ones_v, deg_sh.at[dst_v.at[j]], sem_d[b], add=True)

        if not DIAG_NO_GATHER:
            for b in range(LEAD):
                pltpu.async_copy(y_sh.at[src_v.at[b]], rows_v.at[b], sem_g[b])
        for j in range(NBUF):
            slot(j, j, steady=j >= LEAD)

        @pl.loop(1, NCHUNKS // NBUF)
        def _(g):
            j0 = g * NBUF
            for b in range(NBUF):
                slot(j0 + b, b, steady=True)

        for b in range(NBUF - LEAD, NBUF):
            if not DIAG_NO_AGG:
                pltpu.make_async_copy(
                    rows_v.at[b], agg_sh.at[dst_v.at[0]], sem_s[b]).wait()
            if not DIAG_NO_DEG:
                pltpu.make_async_copy(
                    ones_v, deg_sh.at[dst_v.at[0]], sem_d[b]).wait()

        plsc.subcore_barrier()

        pltpu.sync_copy(agg_sh.at[pl.ds(base, ROWS_PER_SUB)],
                        agg_out.at[cid].at[pl.ds(base, ROWS_PER_SUB)])
        pltpu.sync_copy(deg_sh.at[pl.ds(base, ROWS_PER_SUB)],
                        deg_out.at[cid].at[pl.ds(base, ROWS_PER_SUB)])

    return sc_kernel(y, idx)


def kernel(in_feat, edge_index, W_self, W_neigh, b):
    eidx = edge_index.astype(jnp.int32)
    xf = in_feat.reshape(N_FLAT, 8 * D_IN)
    eye8 = jnp.eye(8, dtype=jnp.float32)
    wbig = jnp.concatenate(
        [jnp.kron(eye8, W_self), jnp.kron(eye8, W_neigh)], axis=1)
    bbig = jnp.concatenate(
        [jnp.tile(b, 8), jnp.zeros((128,), jnp.float32)]).reshape(1, 256)

    zs_flat, y_flat, idx = pl.pallas_call(
        _prep_body,
        grid=(GRID,),
        in_specs=[
            pl.BlockSpec((FLAT_BLOCK, 8 * D_IN), lambda i: (i, 0)),
            pl.BlockSpec((8 * D_IN, 256), lambda i: (0, 0)),
            pl.BlockSpec((1, 256), lambda i: (0, 0)),
            pl.BlockSpec((2, EDGE_BLOCK), lambda i: (0, i)),
        ],
        out_specs=[
            pl.BlockSpec((FLAT_BLOCK, 128), lambda i: (i, 0)),
            pl.BlockSpec((FLAT_BLOCK, 128), lambda i: (i, 0)),
            pl.BlockSpec((2, IDX_BLOCK, CHUNK), lambda i: (0, i, 0)),
        ],
        out_shape=[
            jax.ShapeDtypeStruct((N_FLAT, 128), jnp.float32),
            jax.ShapeDtypeStruct((P_FLAT, 128), jnp.float32),
            jax.ShapeDtypeStruct((2, ROWS_IDX, CHUNK), jnp.int32),
        ],
    )(xf, wbig, bbig, eidx)

    y = y_flat.reshape(N_PAD, D_OUT)
    agg, deg = _sc_segment_sum(y, idx)
    agg_flat = agg.reshape(NC, P_FLAT, 128)
    deg_flat = deg.reshape(NC, P_FLAT, 128)

    out_flat = pl.pallas_call(
        _combine_body,
        grid=(GRID,),
        in_specs=[
            pl.BlockSpec((FLAT_BLOCK, 128), lambda i: (i, 0)),
            pl.BlockSpec((NC, FLAT_BLOCK, 128), lambda i: (0, i, 0)),
            pl.BlockSpec((NC, FLAT_BLOCK, 128), lambda i: (0, i, 0)),
        ],
        out_specs=pl.BlockSpec((FLAT_BLOCK, 128), lambda i: (i, 0)),
        out_shape=jax.ShapeDtypeStruct((N_FLAT, 128), jnp.float32),
    )(zs_flat, agg_flat, deg_flat)
    return out_flat.reshape(N_NODES, D_OUT)


# speedup vs baseline: 25.8348x; 1.0868x over previous
import functools

import jax
import jax.numpy as jnp
from jax import lax
from jax.experimental import pallas as pl
from jax.experimental.pallas import tpu as pltpu
from jax.experimental.pallas import tpu_sc as plsc

N_NODES = 10000
N_EDGES = 320000
D_IN = 128
D_OUT = 16

NC = 2
NS = 16
NW = NC * NS
CHUNK = 128
NBUF = 10
LEAD = 5
DIAG_NO_DEG = False
DIAG_NO_AGG = False
DIAG_NO_GATHER = False
NCHUNKS = 80
ROWS_E = N_EDGES // CHUNK
CHUNKS_LAST = ROWS_E - (NW - 1) * NCHUNKS
N_PAD = 10112
ROWS_PER_SUB = N_PAD // NS

GRID = 10
ROW_BLOCK = 1024
FLAT_BLOCK = ROW_BLOCK * D_OUT // 128
N_FLAT = N_NODES * D_OUT // 128
P_FLAT = N_PAD * D_OUT // 128


def _prep_body(xf_ref, wbig_ref, bbig_ref, zs_ref, y_ref):
    z = jnp.dot(xf_ref[...], wbig_ref[...], preferred_element_type=jnp.float32)
    z = z + bbig_ref[...]
    zs_ref[...] = z[:, :128]
    y_ref[...] = z[:, 128:]


def _combine_body(zs_ref, agg_ref, deg_ref, o_ref):
    agg = agg_ref[0] + agg_ref[1]
    deg = deg_ref[0] + deg_ref[1]
    o_ref[...] = jax.nn.sigmoid(zs_ref[...] + agg / jnp.maximum(deg, 1.0))


def _sc_segment_sum(y, e2):
    mesh = plsc.VectorSubcoreMesh(core_axis_name="c", subcore_axis_name="s")

    @functools.partial(
        pl.kernel,
        out_type=(
            jax.ShapeDtypeStruct((NC, N_PAD, D_OUT), jnp.float32),
            jax.ShapeDtypeStruct((NC, N_PAD, D_OUT), jnp.float32),
        ),
        mesh=mesh,
        compiler_params=pltpu.CompilerParams(use_tc_tiling_on_sc=False),
        scratch_types=[
            pltpu.VMEM((NCHUNKS, CHUNK), jnp.int32),
            pltpu.VMEM((NCHUNKS, CHUNK), jnp.int32),
            pltpu.VMEM((NBUF, CHUNK, D_OUT), jnp.float32),
            pltpu.VMEM((CHUNK, D_OUT), jnp.float32),
            pltpu.VMEM((ROWS_PER_SUB, D_OUT), jnp.float32),
            pltpu.VMEM_SHARED((N_PAD, D_OUT), jnp.float32),
            pltpu.VMEM_SHARED((N_PAD, D_OUT), jnp.float32),
            pltpu.VMEM_SHARED((N_PAD, D_OUT), jnp.float32),
            [pltpu.SemaphoreType.DMA] * NBUF,
            [pltpu.SemaphoreType.DMA] * NBUF,
            [pltpu.SemaphoreType.DMA] * NBUF,
        ],
    )
    def sc_kernel(y_hbm, e_hbm, agg_out, deg_out,
                  src_v, dst_v, rows_v, ones_v, zbuf, y_sh, agg_sh, deg_sh,
                  sem_g, sem_s, sem_d):
        cid = lax.axis_index("c")
        sid = lax.axis_index("s")
        wid = sid * NC + cid
        base = sid * ROWS_PER_SUB

        @pl.loop(0, CHUNK)
        def _(i):
            ones_v.at[i][...] = jnp.full((D_OUT,), 1.0, jnp.float32)

        @pl.loop(0, ROWS_PER_SUB)
        def _(i):
            zbuf.at[i][...] = jnp.zeros((D_OUT,), jnp.float32)

        pltpu.sync_copy(zbuf, agg_sh.at[pl.ds(base, ROWS_PER_SUB)])
        pltpu.sync_copy(zbuf, deg_sh.at[pl.ds(base, ROWS_PER_SUB)])
        pltpu.sync_copy(y_hbm.at[pl.ds(base, ROWS_PER_SUB)],
                        y_sh.at[pl.ds(base, ROWS_PER_SUB)])

        nchunks = jnp.where(wid == NW - 1, CHUNKS_LAST, NCHUNKS)
        ngroups = jnp.where(wid == NW - 1, CHUNKS_LAST // NBUF, NCHUNKS // NBUF)

        @pl.when(wid < NW - 1)
        def _():
            pltpu.sync_copy(e_hbm.at[0].at[pl.ds(wid * NCHUNKS, NCHUNKS)], src_v)
            pltpu.sync_copy(e_hbm.at[1].at[pl.ds(wid * NCHUNKS, NCHUNKS)], dst_v)

        @pl.when(wid == NW - 1)
        def _():
            pltpu.sync_copy(
                e_hbm.at[0].at[pl.ds((NW - 1) * NCHUNKS, CHUNKS_LAST)],
                src_v.at[pl.ds(0, CHUNKS_LAST)])
            pltpu.sync_copy(
                e_hbm.at[1].at[pl.ds((NW - 1) * NCHUNKS, CHUNKS_LAST)],
                dst_v.at[pl.ds(0, CHUNKS_LAST)])

        plsc.subcore_barrier()

        def slot(j, b, steady):
            pb = (b + LEAD) % NBUF
            if steady:
                if not DIAG_NO_AGG:
                    pltpu.make_async_copy(
                        rows_v.at[pb], agg_sh.at[dst_v.at[0]], sem_s[pb]).wait()
                if not DIAG_NO_DEG:
                    pltpu.make_async_copy(
                        ones_v, deg_sh.at[dst_v.at[0]], sem_d[pb]).wait()

            if not DIAG_NO_GATHER:
                if isinstance(j, int):
                    pltpu.async_copy(
                        y_sh.at[src_v.at[j + LEAD]], rows_v.at[pb], sem_g[pb])
                else:
                    @pl.when(j + LEAD < nchunks)
                    def _():
                        pltpu.async_copy(
                            y_sh.at[src_v.at[j + LEAD]], rows_v.at[pb], sem_g[pb])

                pltpu.make_async_copy(
                    y_sh.at[src_v.at[0]], rows_v.at[b], sem_g[b]).wait()
            if not DIAG_NO_AGG:
                pltpu.async_copy(
                    rows_v.at[b], agg_sh.at[dst_v.at[j]], sem_s[b], add=True)
            if not DIAG_NO_DEG:
                pltpu.async_copy(
                    ones_v, deg_sh.at[dst_v.at[j]], sem_d[b], add=True)

        if not DIAG_NO_GATHER:
            for b in range(LEAD):
                pltpu.async_copy(y_sh.at[src_v.at[b]], rows_v.at[b], sem_g[b])
        for j in range(NBUF):
            slot(j, j, steady=j >= LEAD)

        @pl.loop(1, ngroups)
        def _(g):
            j0 = g * NBUF
            for b in range(NBUF):
                slot(j0 + b, b, steady=True)

        for b in range(NBUF - LEAD, NBUF):
            if not DIAG_NO_AGG:
                pltpu.make_async_copy(
                    rows_v.at[b], agg_sh.at[dst_v.at[0]], sem_s[b]).wait()
            if not DIAG_NO_DEG:
                pltpu.make_async_copy(
                    ones_v, deg_sh.at[dst_v.at[0]], sem_d[b]).wait()

        plsc.subcore_barrier()

        pltpu.sync_copy(agg_sh.at[pl.ds(base, ROWS_PER_SUB)],
                        agg_out.at[cid].at[pl.ds(base, ROWS_PER_SUB)])
        pltpu.sync_copy(deg_sh.at[pl.ds(base, ROWS_PER_SUB)],
                        deg_out.at[cid].at[pl.ds(base, ROWS_PER_SUB)])

    return sc_kernel(y, e2)


def kernel(in_feat, edge_index, W_self, W_neigh, b):
    e2 = edge_index.astype(jnp.int32).reshape(2, ROWS_E, CHUNK)
    xf = in_feat.reshape(N_FLAT, 8 * D_IN)
    eye8 = jnp.eye(8, dtype=jnp.float32)
    wbig = jnp.concatenate(
        [jnp.kron(eye8, W_self), jnp.kron(eye8, W_neigh)], axis=1)
    bbig = jnp.concatenate(
        [jnp.tile(b, 8), jnp.zeros((128,), jnp.float32)]).reshape(1, 256)

    zs_flat, y_flat = pl.pallas_call(
        _prep_body,
        grid=(GRID,),
        in_specs=[
            pl.BlockSpec((FLAT_BLOCK, 8 * D_IN), lambda i: (i, 0)),
            pl.BlockSpec((8 * D_IN, 256), lambda i: (0, 0)),
            pl.BlockSpec((1, 256), lambda i: (0, 0)),
        ],
        out_specs=[
            pl.BlockSpec((FLAT_BLOCK, 128), lambda i: (i, 0)),
            pl.BlockSpec((FLAT_BLOCK, 128), lambda i: (i, 0)),
        ],
        out_shape=[
            jax.ShapeDtypeStruct((N_FLAT, 128), jnp.float32),
            jax.ShapeDtypeStruct((P_FLAT, 128), jnp.float32),
        ],
    )(xf, wbig, bbig)

    y = y_flat.reshape(N_PAD, D_OUT)
    agg, deg = _sc_segment_sum(y, e2)
    agg_flat = agg.reshape(NC, P_FLAT, 128)
    deg_flat = deg.reshape(NC, P_FLAT, 128)

    out_flat = pl.pallas_call(
        _combine_body,
        grid=(GRID,),
        in_specs=[
            pl.BlockSpec((FLAT_BLOCK, 128), lambda i: (i, 0)),
            pl.BlockSpec((NC, FLAT_BLOCK, 128), lambda i: (0, i, 0)),
            pl.BlockSpec((NC, FLAT_BLOCK, 128), lambda i: (0, i, 0)),
        ],
        out_specs=pl.BlockSpec((FLAT_BLOCK, 128), lambda i: (i, 0)),
        out_shape=jax.ShapeDtypeStruct((N_FLAT, 128), jnp.float32),
    )(zs_flat, agg_flat, deg_flat)
    return out_flat.reshape(N_NODES, D_OUT)


# speedup vs baseline: 27.9424x; 1.0816x over previous
import functools

import jax
import jax.numpy as jnp
from jax import lax
from jax.experimental import pallas as pl
from jax.experimental.pallas import tpu as pltpu
from jax.experimental.pallas import tpu_sc as plsc

N_NODES = 10000
N_EDGES = 320000
D_IN = 128
D_OUT = 16

NC = 2
NS = 16
NW = NC * NS
CHUNK = 128
NBUF = 10
LEAD = 5
DIAG_NO_DEG = False
DIAG_NO_AGG = False
DIAG_NO_GATHER = False
NCHUNKS = 80
ROWS_E = N_EDGES // CHUNK
CHUNKS_LAST = ROWS_E - (NW - 1) * NCHUNKS
N_PAD = 10112
ROWS_PER_SUB = N_PAD // NS

GRID = 10
ROW_BLOCK = 1024
FLAT_BLOCK = ROW_BLOCK * D_OUT // 128
N_FLAT = N_NODES * D_OUT // 128
P_FLAT = N_PAD * D_OUT // 128


def _prep_body(xf_ref, wcat_ref, bcat_ref, zs_ref, y_ref):
    xf = xf_ref[...]
    zks = [jnp.dot(xf[:, 128 * k:128 * (k + 1)], wcat_ref[...],
                   preferred_element_type=jnp.float32) + bcat_ref[...]
           for k in range(8)]
    zs_ref[...] = jnp.concatenate([zk[:, :D_OUT] for zk in zks], axis=1)
    y_ref[...] = jnp.concatenate([zk[:, D_OUT:] for zk in zks], axis=1)


def _combine_body(zs_ref, agg_ref, deg_ref, o_ref):
    agg = agg_ref[0] + agg_ref[1]
    deg = deg_ref[0] + deg_ref[1]
    o_ref[...] = jax.nn.sigmoid(zs_ref[...] + agg / jnp.maximum(deg, 1.0))


def _sc_segment_sum(y, e2):
    mesh = plsc.VectorSubcoreMesh(core_axis_name="c", subcore_axis_name="s")

    @functools.partial(
        pl.kernel,
        out_type=(
            jax.ShapeDtypeStruct((NC, N_PAD, D_OUT), jnp.float32),
            jax.ShapeDtypeStruct((NC, N_PAD, D_OUT), jnp.float32),
        ),
        mesh=mesh,
        compiler_params=pltpu.CompilerParams(use_tc_tiling_on_sc=False),
        scratch_types=[
            pltpu.VMEM((NCHUNKS, CHUNK), jnp.int32),
            pltpu.VMEM((NCHUNKS, CHUNK), jnp.int32),
            pltpu.VMEM((NBUF, CHUNK, D_OUT), jnp.float32),
            pltpu.VMEM((CHUNK, D_OUT), jnp.float32),
            pltpu.VMEM((ROWS_PER_SUB, D_OUT), jnp.float32),
            pltpu.VMEM_SHARED((N_PAD, D_OUT), jnp.float32),
            pltpu.VMEM_SHARED((N_PAD, D_OUT), jnp.float32),
            pltpu.VMEM_SHARED((N_PAD, D_OUT), jnp.float32),
            [pltpu.SemaphoreType.DMA] * NBUF,
            [pltpu.SemaphoreType.DMA] * NBUF,
            [pltpu.SemaphoreType.DMA] * NBUF,
        ],
    )
    def sc_kernel(y_hbm, e_hbm, agg_out, deg_out,
                  src_v, dst_v, rows_v, ones_v, zbuf, y_sh, agg_sh, deg_sh,
                  sem_g, sem_s, sem_d):
        cid = lax.axis_index("c")
        sid = lax.axis_index("s")
        wid = sid * NC + cid
        base = sid * ROWS_PER_SUB

        @pl.loop(0, CHUNK)
        def _(i):
            ones_v.at[i][...] = jnp.full((D_OUT,), 1.0, jnp.float32)

        @pl.loop(0, ROWS_PER_SUB)
        def _(i):
            zbuf.at[i][...] = jnp.zeros((D_OUT,), jnp.float32)

        pltpu.sync_copy(zbuf, agg_sh.at[pl.ds(base, ROWS_PER_SUB)])
        pltpu.sync_copy(zbuf, deg_sh.at[pl.ds(base, ROWS_PER_SUB)])
        pltpu.sync_copy(y_hbm.at[pl.ds(base, ROWS_PER_SUB)],
                        y_sh.at[pl.ds(base, ROWS_PER_SUB)])

        nchunks = jnp.where(wid == NW - 1, CHUNKS_LAST, NCHUNKS)
        ngroups = jnp.where(wid == NW - 1, CHUNKS_LAST // NBUF, NCHUNKS // NBUF)

        @pl.when(wid < NW - 1)
        def _():
            pltpu.sync_copy(e_hbm.at[0].at[pl.ds(wid * NCHUNKS, NCHUNKS)], src_v)
            pltpu.sync_copy(e_hbm.at[1].at[pl.ds(wid * NCHUNKS, NCHUNKS)], dst_v)

        @pl.when(wid == NW - 1)
        def _():
            pltpu.sync_copy(
                e_hbm.at[0].at[pl.ds((NW - 1) * NCHUNKS, CHUNKS_LAST)],
                src_v.at[pl.ds(0, CHUNKS_LAST)])
            pltpu.sync_copy(
                e_hbm.at[1].at[pl.ds((NW - 1) * NCHUNKS, CHUNKS_LAST)],
                dst_v.at[pl.ds(0, CHUNKS_LAST)])

        plsc.subcore_barrier()

        def slot(j, b, steady):
            pb = (b + LEAD) % NBUF
            if steady:
                if not DIAG_NO_AGG:
                    pltpu.make_async_copy(
                        rows_v.at[pb], agg_sh.at[dst_v.at[0]], sem_s[pb]).wait()
                if not DIAG_NO_DEG:
                    pltpu.make_async_copy(
                        ones_v, deg_sh.at[dst_v.at[0]], sem_d[pb]).wait()

            if not DIAG_NO_GATHER:
                if isinstance(j, int):
                    pltpu.async_copy(
                        y_sh.at[src_v.at[j + LEAD]], rows_v.at[pb], sem_g[pb])
                else:
                    @pl.when(j + LEAD < nchunks)
                    def _():
                        pltpu.async_copy(
                            y_sh.at[src_v.at[j + LEAD]], rows_v.at[pb], sem_g[pb])

                pltpu.make_async_copy(
                    y_sh.at[src_v.at[0]], rows_v.at[b], sem_g[b]).wait()
            if not DIAG_NO_AGG:
                pltpu.async_copy(
                    rows_v.at[b], agg_sh.at[dst_v.at[j]], sem_s[b], add=True)
            if not DIAG_NO_DEG:
                pltpu.async_copy(
                    ones_v, deg_sh.at[dst_v.at[j]], sem_d[b], add=True)

        if not DIAG_NO_GATHER:
            for b in range(LEAD):
                pltpu.async_copy(y_sh.at[src_v.at[b]], rows_v.at[b], sem_g[b])
        for j in range(NBUF):
            slot(j, j, steady=j >= LEAD)

        @pl.loop(1, ngroups)
        def _(g):
            j0 = g * NBUF
            for b in range(NBUF):
                slot(j0 + b, b, steady=True)

        for b in range(NBUF - LEAD, NBUF):
            if not DIAG_NO_AGG:
                pltpu.make_async_copy(
                    rows_v.at[b], agg_sh.at[dst_v.at[0]], sem_s[b]).wait()
            if not DIAG_NO_DEG:
                pltpu.make_async_copy(
                    ones_v, deg_sh.at[dst_v.at[0]], sem_d[b]).wait()

        plsc.subcore_barrier()

        pltpu.sync_copy(agg_sh.at[pl.ds(base, ROWS_PER_SUB)],
                        agg_out.at[cid].at[pl.ds(base, ROWS_PER_SUB)])
        pltpu.sync_copy(deg_sh.at[pl.ds(base, ROWS_PER_SUB)],
                        deg_out.at[cid].at[pl.ds(base, ROWS_PER_SUB)])

    return sc_kernel(y, e2)


def kernel(in_feat, edge_index, W_self, W_neigh, b):
    e2 = edge_index.astype(jnp.int32).reshape(2, ROWS_E, CHUNK)
    xf = in_feat.reshape(N_FLAT, 8 * D_IN)
    wcat = jnp.concatenate([W_self, W_neigh], axis=1)
    bcat = jnp.concatenate(
        [b, jnp.zeros((D_OUT,), jnp.float32)]).reshape(1, 2 * D_OUT)

    zs_flat, y_flat = pl.pallas_call(
        _prep_body,
        grid=(GRID,),
        in_specs=[
            pl.BlockSpec((FLAT_BLOCK, 8 * D_IN), lambda i: (i, 0)),
            pl.BlockSpec((D_IN, 2 * D_OUT), lambda i: (0, 0)),
            pl.BlockSpec((1, 2 * D_OUT), lambda i: (0, 0)),
        ],
        out_specs=[
            pl.BlockSpec((FLAT_BLOCK, 128), lambda i: (i, 0)),
            pl.BlockSpec((FLAT_BLOCK, 128), lambda i: (i, 0)),
        ],
        out_shape=[
            jax.ShapeDtypeStruct((N_FLAT, 128), jnp.float32),
            jax.ShapeDtypeStruct((P_FLAT, 128), jnp.float32),
        ],
    )(xf, wcat, bcat)

    y = y_flat.reshape(N_PAD, D_OUT)
    agg, deg = _sc_segment_sum(y, e2)
    agg_flat = agg.reshape(NC, P_FLAT, 128)
    deg_flat = deg.reshape(NC, P_FLAT, 128)

    out_flat = pl.pallas_call(
        _combine_body,
        grid=(GRID,),
        in_specs=[
            pl.BlockSpec((FLAT_BLOCK, 128), lambda i: (i, 0)),
            pl.BlockSpec((NC, FLAT_BLOCK, 128), lambda i: (0, i, 0)),
            pl.BlockSpec((NC, FLAT_BLOCK, 128), lambda i: (0, i, 0)),
        ],
        out_specs=pl.BlockSpec((FLAT_BLOCK, 128), lambda i: (i, 0)),
        out_shape=jax.ShapeDtypeStruct((N_FLAT, 128), jnp.float32),
    )(zs_flat, agg_flat, deg_flat)
    return out_flat.reshape(N_NODES, D_OUT)
